# baseline (speedup 1.0000x reference)
import functools

import jax
import jax.numpy as jnp
from jax.experimental import pallas as pl
from jax.experimental.pallas import tpu as pltpu

_P = jax.lax.Precision.DEFAULT


def _dot(a, b):
    return jax.lax.dot_general(
        a, b, (((1,), (0,)), ((), ())),
        preferred_element_type=jnp.float32, precision=_P)


def _fused(nblk1, gene_ref, adj_ref, x_ref, ws_ref, bs_ref, wf_ref,
           bf_ref, w1_ref, b1_ref, w2_ref, b2_ref, w3_ref, b3_ref,
           out_ref, hf_ref):
    i = pl.program_id(0)
    bm1 = gene_ref.shape[0]

    @pl.when(i < nblk1)
    def _phase0():
        ax = _dot(gene_ref[...], x_ref[...])
        h = jnp.tanh(_dot(ax, ws_ref[...]) + bs_ref[...])
        j = jnp.minimum(i, nblk1 - 1)
        hf_ref[pl.ds(j * bm1, bm1), :] = _dot(h, wf_ref[...])

    @pl.when(i >= nblk1)
    def _phase1():
        acc = _dot(adj_ref[...], hf_ref[...])
        h = jnp.tanh(acc + bf_ref[...])
        h = jnp.tanh(_dot(h, w1_ref[...]) + b1_ref[...])
        h = jnp.tanh(_dot(h, w2_ref[...]) + b2_ref[...])
        out_ref[...] = _dot(h, w3_ref[...]) + b3_ref[...]


def kernel(x, adj, gene_adj, W_s, b_s, W_f, b_f, W1, b1, W2, b2, W3, b3):
    n, f = x.shape
    f1 = W1.shape[1]
    f2 = W2.shape[1]
    nc = W3.shape[1]
    bm1 = 400 if n % 400 == 0 else (8 if n % 8 == 0 else n)
    bm2 = 200 if n % 200 == 0 else (8 if n % 8 == 0 else n)
    nblk1 = n // bm1
    nblk2 = n // bm2

    def _const(shape):
        return pl.BlockSpec(shape, lambda i: (0, 0))

    body = functools.partial(_fused, nblk1)

    out = pl.pallas_call(
        body,
        grid=(nblk1 + nblk2,),
        in_specs=[
            pl.BlockSpec((bm1, n), lambda i: (jnp.minimum(i, nblk1 - 1), 0)),
            pl.BlockSpec((bm2, n), lambda i: (jnp.maximum(i - nblk1, 0), 0)),
            _const((n, f)),
            _const((f, f)),
            _const((1, f)),
            _const((f, f)),
            _const((1, f)),
            _const((f, f1)),
            _const((1, f1)),
            _const((f1, f2)),
            _const((1, f2)),
            _const((f2, nc)),
            _const((1, nc)),
        ],
        out_specs=pl.BlockSpec(
            (bm2, nc), lambda i: (jnp.maximum(i - nblk1, 0), 0)),
        out_shape=jax.ShapeDtypeStruct((n, nc), jnp.float32),
        scratch_shapes=[pltpu.VMEM((n, f), jnp.float32)],
        compiler_params=pltpu.CompilerParams(
            dimension_semantics=("arbitrary",),
            vmem_limit_bytes=62 * 1024 * 1024,
        ),
    )(gene_adj, adj, x, W_s, b_s.reshape(1, f), W_f, b_f.reshape(1, f),
      W1, b1.reshape(1, f1), W2, b2.reshape(1, f2), W3, b3.reshape(1, nc))
    return out
